# Initial kernel scaffold; baseline (speedup 1.0000x reference)
import jax
import jax.numpy as jnp
from jax.experimental import pallas as pl


def kernel(Jt_ind, Jt_val, f, N, W1, b1, W2, b2, Wc, bc):
    raise NotImplementedError("write your pallas kernel here")


# speedup vs baseline: 13.3216x; 13.3216x over previous
import jax
import jax.numpy as jnp
from jax import lax
from jax.experimental import pallas as pl
from jax.experimental.pallas import tpu as pltpu
from jax.experimental.pallas import tpu_sc as plsc

NC = 2
NS = 16
L = 16
NW = NC * NS

_NEG_INF = float("-inf")


def _take(x, idx):
  return jnp.take_along_axis(x, idx, axis=0, mode="promise_in_bounds")


def _seg_scatter(ref, k, v, op, ident):
  ks, vs = plsc.sort_key_val(k, v)
  lane = lax.iota(jnp.int32, L)
  for s in (1, 2, 4, 8):
    idx = jnp.maximum(lane - s, 0)
    same = jnp.logical_and(_take(ks, idx) == ks, lane >= s)
    vs = op(vs, jnp.where(same, _take(vs, idx), ident))
  nxt = _take(ks, jnp.minimum(lane + 1, L - 1))
  last = jnp.logical_or(nxt != ks, lane == L - 1)
  cur = plsc.load_gather(ref, [ks], mask=last)
  plsc.store_scatter(ref, [ks], op(cur, vs), mask=last)


def _fill(ref, n, value):
  splat = jnp.full((L,), value, jnp.float32)

  def body(u, _):
    ref[pl.ds(u * L, L)] = splat
    return 0

  lax.fori_loop(0, n // L, body, 0)


def _ewise(dst, src, n, op):
  def body(u, _):
    sl = pl.ds(u * L, L)
    dst[sl] = op(dst[sl], src[sl])
    return 0

  lax.fori_loop(0, n // L, body, 0)


def _combine_and_emit(loc, shared, acc, tmp, part_hbm, cid, sid, npad, op):
  s_sz = npad // NS
  pltpu.sync_copy(loc, shared.at[sid])
  plsc.subcore_barrier()
  base = sid * s_sz
  pltpu.sync_copy(shared.at[0, pl.ds(base, s_sz)], acc)
  for t in range(1, NS):
    pltpu.sync_copy(shared.at[t, pl.ds(base, s_sz)], tmp.at[pl.ds(0, s_sz)])
    _ewise(acc, tmp, s_sz, op)
  pltpu.sync_copy(acc, part_hbm.at[pl.ds(cid * npad + base, s_sz)])


def _merge_parts(part_hbm, dst, tmp, npad, op):
  pltpu.sync_copy(part_hbm.at[pl.ds(0, npad)], dst)
  pltpu.sync_copy(part_hbm.at[pl.ds(npad, npad)], tmp)
  _ewise(dst, tmp, npad, op)


def _wid(cid, sid):
  return sid * NC + cid


def _sc_mesh():
  return plsc.VectorSubcoreMesh(core_axis_name="c", subcore_axis_name="s")


def _sc_segsum_abs(i32e, val, npad, ew):
  def body(i_hbm, v_hbm, part_hbm, ibuf, vbuf, loc, acc, tmp, shared):
    cid = lax.axis_index("c")
    sid = lax.axis_index("s")
    base = _wid(cid, sid) * ew
    pltpu.sync_copy(i_hbm.at[pl.ds(base, ew)], ibuf)
    pltpu.sync_copy(v_hbm.at[pl.ds(base, ew)], vbuf)
    _fill(loc, npad, 0.0)

    def eb(b, _):
      sl = pl.ds(b * L, L)
      _seg_scatter(loc, ibuf[sl], jnp.abs(vbuf[sl]), jnp.add, 0.0)
      return 0

    lax.fori_loop(0, ew // L, eb, 0)
    _combine_and_emit(loc, shared, acc, tmp, part_hbm, cid, sid, npad, jnp.add)

  return pl.kernel(
      body,
      out_type=jax.ShapeDtypeStruct((NC * npad,), jnp.float32),
      mesh=_sc_mesh(),
      compiler_params=pltpu.CompilerParams(needs_layout_passes=False),
      scratch_types=[
          pltpu.VMEM((ew,), jnp.int32),
          pltpu.VMEM((ew,), jnp.float32),
          pltpu.VMEM((npad,), jnp.float32),
          pltpu.VMEM((npad // NS,), jnp.float32),
          pltpu.VMEM((npad,), jnp.float32),
          pltpu.VMEM_SHARED((NS, npad), jnp.float32),
      ],
  )(i32e, val)


def _sc_segmax(i32e, e, npad, ew):
  def body(i_hbm, e_hbm, part_hbm, ibuf, ebuf, loc, acc, tmp, shared):
    cid = lax.axis_index("c")
    sid = lax.axis_index("s")
    base = _wid(cid, sid) * ew
    pltpu.sync_copy(i_hbm.at[pl.ds(base, ew)], ibuf)
    pltpu.sync_copy(e_hbm.at[pl.ds(base, ew)], ebuf)
    _fill(loc, npad, _NEG_INF)

    def eb(b, _):
      sl = pl.ds(b * L, L)
      _seg_scatter(loc, ibuf[sl], ebuf[sl], jnp.maximum, _NEG_INF)
      return 0

    lax.fori_loop(0, ew // L, eb, 0)
    _combine_and_emit(loc, shared, acc, tmp, part_hbm, cid, sid, npad,
                      jnp.maximum)

  return pl.kernel(
      body,
      out_type=jax.ShapeDtypeStruct((NC * npad,), jnp.float32),
      mesh=_sc_mesh(),
      compiler_params=pltpu.CompilerParams(needs_layout_passes=False),
      scratch_types=[
          pltpu.VMEM((ew,), jnp.int32),
          pltpu.VMEM((ew,), jnp.float32),
          pltpu.VMEM((npad,), jnp.float32),
          pltpu.VMEM((npad // NS,), jnp.float32),
          pltpu.VMEM((npad,), jnp.float32),
          pltpu.VMEM_SHARED((NS, npad), jnp.float32),
      ],
  )(i32e, e)


def _sc_exp_segsum(i32e, e, mx_part, npad, ew):
  def body(i_hbm, e_hbm, mx_hbm, p_hbm, part_hbm, ibuf, ebuf, m, loc, acc,
           tmp, shared):
    cid = lax.axis_index("c")
    sid = lax.axis_index("s")
    base = _wid(cid, sid) * ew
    _merge_parts(mx_hbm, m, tmp, npad, jnp.maximum)
    pltpu.sync_copy(i_hbm.at[pl.ds(base, ew)], ibuf)
    pltpu.sync_copy(e_hbm.at[pl.ds(base, ew)], ebuf)
    _fill(loc, npad, 0.0)

    def eb(b, _):
      sl = pl.ds(b * L, L)
      k = ibuf[sl]
      mg = plsc.load_gather(m, [k])
      p = jnp.exp(ebuf[sl] - mg)
      ebuf[sl] = p
      _seg_scatter(loc, k, p, jnp.add, 0.0)
      return 0

    lax.fori_loop(0, ew // L, eb, 0)
    pltpu.sync_copy(ebuf, p_hbm.at[pl.ds(base, ew)])
    _combine_and_emit(loc, shared, acc, tmp, part_hbm, cid, sid, npad, jnp.add)

  return pl.kernel(
      body,
      out_type=[
          jax.ShapeDtypeStruct((ew * NW,), jnp.float32),
          jax.ShapeDtypeStruct((NC * npad,), jnp.float32),
      ],
      mesh=_sc_mesh(),
      compiler_params=pltpu.CompilerParams(needs_layout_passes=False),
      scratch_types=[
          pltpu.VMEM((ew,), jnp.int32),
          pltpu.VMEM((ew,), jnp.float32),
          pltpu.VMEM((npad,), jnp.float32),
          pltpu.VMEM((npad,), jnp.float32),
          pltpu.VMEM((npad // NS,), jnp.float32),
          pltpu.VMEM((npad,), jnp.float32),
          pltpu.VMEM_SHARED((NS, npad), jnp.float32),
      ],
  )(i32e, e, mx_part)


def _sc_final(i32e, p, den_part, npad, ew):
  def body(i_hbm, p_hbm, den_hbm, a_hbm, ibuf, pbuf, d, tmp):
    cid = lax.axis_index("c")
    sid = lax.axis_index("s")
    base = _wid(cid, sid) * ew
    _merge_parts(den_hbm, d, tmp, npad, jnp.add)
    pltpu.sync_copy(i_hbm.at[pl.ds(base, ew)], ibuf)
    pltpu.sync_copy(p_hbm.at[pl.ds(base, ew)], pbuf)

    def eb(b, _):
      sl = pl.ds(b * L, L)
      dg = plsc.load_gather(d, [ibuf[sl]])
      pbuf[sl] = pbuf[sl] / dg
      return 0

    lax.fori_loop(0, ew // L, eb, 0)
    pltpu.sync_copy(pbuf, a_hbm.at[pl.ds(base, ew)])

  return pl.kernel(
      body,
      out_type=jax.ShapeDtypeStruct((ew * NW,), jnp.float32),
      mesh=_sc_mesh(),
      compiler_params=pltpu.CompilerParams(needs_layout_passes=False),
      scratch_types=[
          pltpu.VMEM((ew,), jnp.int32),
          pltpu.VMEM((ew,), jnp.float32),
          pltpu.VMEM((npad,), jnp.float32),
          pltpu.VMEM((npad,), jnp.float32),
      ],
  )(i32e, p, den_part)


def _sc_gather(i32e, j32e, val, g, h, deg_part, npad, ew):
  h1 = g.shape[1]
  sup = 2000
  nsup = ew // sup
  qs = []
  off = 0
  while off < sup:
    qn = min(128, sup - off)
    qs.append((off, qn))
    off += qn

  def body(i_hbm, j_hbm, v_hbm, g_hbm, h_hbm, dp_hbm, s_hbm, n_hbm,
           ibuf, jbuf, vbuf, nbuf, deg, tmp, sbuf, sem):
    cid = lax.axis_index("c")
    sid = lax.axis_index("s")
    base = _wid(cid, sid) * ew
    _merge_parts(dp_hbm, deg, tmp, npad, jnp.add)
    for sc in range(nsup):
      cb = base + sc * sup
      pltpu.sync_copy(i_hbm.at[pl.ds(cb, sup)], ibuf)
      pltpu.sync_copy(j_hbm.at[pl.ds(cb, sup)], jbuf)
      pltpu.sync_copy(v_hbm.at[pl.ds(cb, sup)], vbuf)
      descs = [
          pltpu.async_copy(g_hbm.at[ibuf.at[pl.ds(qo, qn)]],
                           sbuf.at[pl.ds(qo, qn)], sem)
          for qo, qn in qs
      ]
      for dsc in descs:
        dsc.wait()
      descs = [
          pltpu.async_copy(h_hbm.at[jbuf.at[pl.ds(qo, qn)]],
                           sbuf.at[pl.ds(qo, qn)], sem, add=True)
          for qo, qn in qs
      ]
      for dsc in descs:
        dsc.wait()

      def nb(b, _):
        sl = pl.ds(b * L, L)
        dg = plsc.load_gather(deg, [ibuf[sl]])
        nbuf[pl.ds(sc * sup + b * L, L)] = jnp.abs(vbuf[sl]) / dg
        return 0

      lax.fori_loop(0, sup // L, nb, 0)
      pltpu.sync_copy(sbuf, s_hbm.at[pl.ds(cb, sup)])
    pltpu.sync_copy(nbuf, n_hbm.at[pl.ds(base, ew)])

  return pl.kernel(
      body,
      out_type=[
          jax.ShapeDtypeStruct((ew * NW, h1), jnp.float32),
          jax.ShapeDtypeStruct((ew * NW,), jnp.float32),
      ],
      mesh=_sc_mesh(),
      compiler_params=pltpu.CompilerParams(
          needs_layout_passes=False, use_tc_tiling_on_sc=False),
      scratch_types=[
          pltpu.VMEM((sup,), jnp.int32),
          pltpu.VMEM((sup,), jnp.int32),
          pltpu.VMEM((sup,), jnp.float32),
          pltpu.VMEM((ew,), jnp.float32),
          pltpu.VMEM((npad,), jnp.float32),
          pltpu.VMEM((npad,), jnp.float32),
          pltpu.VMEM((sup, h1), jnp.float32),
          pltpu.SemaphoreType.DMA,
      ],
  )(i32e, j32e, val, g, h, deg_part)


def _tc_project(f, W1, b1):
  n, d = f.shape
  h1 = W1.shape[1]
  W1cat = jnp.concatenate([W1[:d], W1[d:]], axis=1)
  b1h = (jnp.concatenate([b1, b1]) * 0.5).reshape(1, 2 * h1)

  def body(f_ref, w_ref, b_ref, g_ref, h_ref):
    gh = jnp.dot(f_ref[...], w_ref[...], preferred_element_type=jnp.float32)
    gh = gh + b_ref[...]
    g_ref[...] = gh[:, :h1]
    h_ref[...] = gh[:, h1:]

  return pl.pallas_call(
      body,
      out_shape=[
          jax.ShapeDtypeStruct((n, h1), jnp.float32),
          jax.ShapeDtypeStruct((n, h1), jnp.float32),
      ],
  )(f, W1cat, b1h)


def _tc_mlp(s, norm1, W2, b2, Wc):
  ecnt, h1 = s.shape
  h2 = W2.shape[0]
  rows = 3200
  grid = ecnt // rows
  w2a = W2[:h1, :]
  w2n = W2[h1:, :].reshape(1, h2)
  b2r = b2.reshape(1, h2)
  wcr = Wc.reshape(1, h2)

  def body(s_ref, n_ref, w2a_ref, w2n_ref, b2_ref, wc_ref, e_ref):
    sv = s_ref[...]
    x1 = jnp.where(sv >= 0, sv, 0.1 * sv)
    pre = jnp.dot(x1, w2a_ref[...], preferred_element_type=jnp.float32)
    pre = pre + n_ref[...] * w2n_ref[...] + b2_ref[...]
    x2 = jnp.where(pre >= 0, pre, 0.1 * pre)
    e_ref[...] = jnp.sum(x2 * wc_ref[...], axis=1, keepdims=True)

  return pl.pallas_call(
      body,
      grid=(grid,),
      in_specs=[
          pl.BlockSpec((rows, h1), lambda t: (t, 0)),
          pl.BlockSpec((rows, 1), lambda t: (t, 0)),
          pl.BlockSpec((h1, h2), lambda t: (0, 0)),
          pl.BlockSpec((1, h2), lambda t: (0, 0)),
          pl.BlockSpec((1, h2), lambda t: (0, 0)),
          pl.BlockSpec((1, h2), lambda t: (0, 0)),
      ],
      out_specs=pl.BlockSpec((rows, 1), lambda t: (t, 0)),
      out_shape=jax.ShapeDtypeStruct((ecnt, 1), jnp.float32),
  )(s, norm1, w2a, w2n, b2r, wcr)


def kernel(Jt_ind, Jt_val, f, N, W1, b1, W2, b2, Wc, bc):
  del N, bc
  ecnt = Jt_ind.shape[1]
  n = f.shape[0]
  npad = ((n + 255) // 256) * 256
  ew = ecnt // NW
  i32e = Jt_ind[0].astype(jnp.int32)
  j32e = Jt_ind[1].astype(jnp.int32)
  val = Jt_val.astype(jnp.float32)

  deg_part = _sc_segsum_abs(i32e, val, npad, ew)
  g, h = _tc_project(f, W1, b1)
  s, norm = _sc_gather(i32e, j32e, val, g, h, deg_part, npad, ew)
  e = _tc_mlp(s, norm.reshape(ecnt, 1), W2, b2, Wc).reshape(ecnt)
  mx_part = _sc_segmax(i32e, e, npad, ew)
  p, den_part = _sc_exp_segsum(i32e, e, mx_part, npad, ew)
  a = _sc_final(i32e, p, den_part, npad, ew)
  return a


# speedup vs baseline: 36.0911x; 2.7092x over previous
import jax
import jax.numpy as jnp
from jax import lax
from jax.experimental import pallas as pl
from jax.experimental.pallas import tpu as pltpu
from jax.experimental.pallas import tpu_sc as plsc

NC = 2
NS = 16
L = 16
NW = NC * NS

_NEG_INF = float("-inf")


def _take(x, idx):
  return jnp.take_along_axis(x, idx, axis=0, mode="promise_in_bounds")


def _seg_scatter(ref, k, v, op, ident):
  ks, vs = plsc.sort_key_val(k, v)
  lane = lax.iota(jnp.int32, L)
  for s in (1, 2, 4, 8):
    idx = jnp.maximum(lane - s, 0)
    same = jnp.logical_and(_take(ks, idx) == ks, lane >= s)
    vs = op(vs, jnp.where(same, _take(vs, idx), ident))
  nxt = _take(ks, jnp.minimum(lane + 1, L - 1))
  last = jnp.logical_or(nxt != ks, lane == L - 1)
  cur = plsc.load_gather(ref, [ks], mask=last)
  plsc.store_scatter(ref, [ks], op(cur, vs), mask=last)


def _fill(ref, n, value):
  splat = jnp.full((L,), value, jnp.float32)

  def body(u, _):
    ref[pl.ds(u * L, L)] = splat
    return 0

  lax.fori_loop(0, n // L, body, 0)


def _ewise(dst, src, n, op):
  def body(u, _):
    sl = pl.ds(u * L, L)
    dst[sl] = op(dst[sl], src[sl])
    return 0

  lax.fori_loop(0, n // L, body, 0)


def _combine_and_emit(loc, shared, acc, tmp, part_hbm, cid, sid, npad, op):
  s_sz = npad // NS
  pltpu.sync_copy(loc, shared.at[sid])
  plsc.subcore_barrier()
  base = sid * s_sz
  pltpu.sync_copy(shared.at[0, pl.ds(base, s_sz)], acc)
  for t in range(1, NS):
    pltpu.sync_copy(shared.at[t, pl.ds(base, s_sz)], tmp.at[pl.ds(0, s_sz)])
    _ewise(acc, tmp, s_sz, op)
  pltpu.sync_copy(acc, part_hbm.at[pl.ds(cid * npad + base, s_sz)])


def _merge_parts(part_hbm, dst, tmp, npad, op):
  pltpu.sync_copy(part_hbm.at[pl.ds(0, npad)], dst)
  pltpu.sync_copy(part_hbm.at[pl.ds(npad, npad)], tmp)
  _ewise(dst, tmp, npad, op)


def _wid(cid, sid):
  return sid * NC + cid


def _sc_mesh():
  return plsc.VectorSubcoreMesh(core_axis_name="c", subcore_axis_name="s")


def _sc_segsum_abs(i32e, val, npad, ew):
  def body(i_hbm, v_hbm, part_hbm, ibuf, vbuf, loc, acc, tmp, shared):
    cid = lax.axis_index("c")
    sid = lax.axis_index("s")
    base = _wid(cid, sid) * ew
    pltpu.sync_copy(i_hbm.at[pl.ds(base, ew)], ibuf)
    pltpu.sync_copy(v_hbm.at[pl.ds(base, ew)], vbuf)
    _fill(loc, npad, 0.0)

    def eb(b, _):
      sl = pl.ds(b * L, L)
      plsc.addupdate_scatter(loc, [ibuf[sl]], jnp.abs(vbuf[sl]))
      return 0

    lax.fori_loop(0, ew // L, eb, 0)
    _combine_and_emit(loc, shared, acc, tmp, part_hbm, cid, sid, npad, jnp.add)

  return pl.kernel(
      body,
      out_type=jax.ShapeDtypeStruct((NC * npad,), jnp.float32),
      mesh=_sc_mesh(),
      compiler_params=pltpu.CompilerParams(needs_layout_passes=False),
      scratch_types=[
          pltpu.VMEM((ew,), jnp.int32),
          pltpu.VMEM((ew,), jnp.float32),
          pltpu.VMEM((npad,), jnp.float32),
          pltpu.VMEM((npad // NS,), jnp.float32),
          pltpu.VMEM((npad,), jnp.float32),
          pltpu.VMEM_SHARED((NS, npad), jnp.float32),
      ],
  )(i32e, val)


def _sc_segmax(i32e, e, npad, ew):
  def body(i_hbm, e_hbm, part_hbm, ibuf, ebuf, loc, acc, tmp, shared):
    cid = lax.axis_index("c")
    sid = lax.axis_index("s")
    base = _wid(cid, sid) * ew
    pltpu.sync_copy(i_hbm.at[pl.ds(base, ew)], ibuf)
    pltpu.sync_copy(e_hbm.at[pl.ds(base, ew)], ebuf)
    _fill(loc, npad, _NEG_INF)

    def eb(b, _):
      sl = pl.ds(b * L, L)
      _seg_scatter(loc, ibuf[sl], ebuf[sl], jnp.maximum, _NEG_INF)
      return 0

    lax.fori_loop(0, ew // L, eb, 0)
    _combine_and_emit(loc, shared, acc, tmp, part_hbm, cid, sid, npad,
                      jnp.maximum)

  return pl.kernel(
      body,
      out_type=jax.ShapeDtypeStruct((NC * npad,), jnp.float32),
      mesh=_sc_mesh(),
      compiler_params=pltpu.CompilerParams(needs_layout_passes=False),
      scratch_types=[
          pltpu.VMEM((ew,), jnp.int32),
          pltpu.VMEM((ew,), jnp.float32),
          pltpu.VMEM((npad,), jnp.float32),
          pltpu.VMEM((npad // NS,), jnp.float32),
          pltpu.VMEM((npad,), jnp.float32),
          pltpu.VMEM_SHARED((NS, npad), jnp.float32),
      ],
  )(i32e, e)


def _sc_exp_segsum(i32e, e, mx_part, npad, ew):
  def body(i_hbm, e_hbm, mx_hbm, p_hbm, part_hbm, ibuf, ebuf, m, loc, acc,
           tmp, shared):
    cid = lax.axis_index("c")
    sid = lax.axis_index("s")
    base = _wid(cid, sid) * ew
    _merge_parts(mx_hbm, m, tmp, npad, jnp.maximum)
    pltpu.sync_copy(i_hbm.at[pl.ds(base, ew)], ibuf)
    pltpu.sync_copy(e_hbm.at[pl.ds(base, ew)], ebuf)
    _fill(loc, npad, 0.0)

    def eb(b, _):
      sl = pl.ds(b * L, L)
      k = ibuf[sl]
      mg = plsc.load_gather(m, [k])
      p = jnp.exp(ebuf[sl] - mg)
      ebuf[sl] = p
      plsc.addupdate_scatter(loc, [k], p)
      return 0

    lax.fori_loop(0, ew // L, eb, 0)
    pltpu.sync_copy(ebuf, p_hbm.at[pl.ds(base, ew)])
    _combine_and_emit(loc, shared, acc, tmp, part_hbm, cid, sid, npad, jnp.add)

  return pl.kernel(
      body,
      out_type=[
          jax.ShapeDtypeStruct((ew * NW,), jnp.float32),
          jax.ShapeDtypeStruct((NC * npad,), jnp.float32),
      ],
      mesh=_sc_mesh(),
      compiler_params=pltpu.CompilerParams(needs_layout_passes=False),
      scratch_types=[
          pltpu.VMEM((ew,), jnp.int32),
          pltpu.VMEM((ew,), jnp.float32),
          pltpu.VMEM((npad,), jnp.float32),
          pltpu.VMEM((npad,), jnp.float32),
          pltpu.VMEM((npad // NS,), jnp.float32),
          pltpu.VMEM((npad,), jnp.float32),
          pltpu.VMEM_SHARED((NS, npad), jnp.float32),
      ],
  )(i32e, e, mx_part)


def _sc_final(i32e, p, den_part, npad, ew):
  def body(i_hbm, p_hbm, den_hbm, a_hbm, ibuf, pbuf, d, tmp):
    cid = lax.axis_index("c")
    sid = lax.axis_index("s")
    base = _wid(cid, sid) * ew
    _merge_parts(den_hbm, d, tmp, npad, jnp.add)
    pltpu.sync_copy(i_hbm.at[pl.ds(base, ew)], ibuf)
    pltpu.sync_copy(p_hbm.at[pl.ds(base, ew)], pbuf)

    def eb(b, _):
      sl = pl.ds(b * L, L)
      dg = plsc.load_gather(d, [ibuf[sl]])
      pbuf[sl] = pbuf[sl] / dg
      return 0

    lax.fori_loop(0, ew // L, eb, 0)
    pltpu.sync_copy(pbuf, a_hbm.at[pl.ds(base, ew)])

  return pl.kernel(
      body,
      out_type=jax.ShapeDtypeStruct((ew * NW,), jnp.float32),
      mesh=_sc_mesh(),
      compiler_params=pltpu.CompilerParams(needs_layout_passes=False),
      scratch_types=[
          pltpu.VMEM((ew,), jnp.int32),
          pltpu.VMEM((ew,), jnp.float32),
          pltpu.VMEM((npad,), jnp.float32),
          pltpu.VMEM((npad,), jnp.float32),
      ],
  )(i32e, p, den_part)


def _sc_gather(i32e, j32e, val, g, h, deg_part, npad, ew):
  h1 = g.shape[1]
  sup = 2000
  nsup = ew // sup
  qs = []
  off = 0
  while off < sup:
    qn = min(128, sup - off)
    qs.append((off, qn))
    off += qn

  def body(i_hbm, j_hbm, v_hbm, g_hbm, h_hbm, dp_hbm, s_hbm, n_hbm,
           ibuf, jbuf, vbuf, nbuf, deg, tmp, sbuf, sem):
    cid = lax.axis_index("c")
    sid = lax.axis_index("s")
    base = _wid(cid, sid) * ew
    _merge_parts(dp_hbm, deg, tmp, npad, jnp.add)
    for sc in range(nsup):
      cb = base + sc * sup
      pltpu.sync_copy(i_hbm.at[pl.ds(cb, sup)], ibuf)
      pltpu.sync_copy(j_hbm.at[pl.ds(cb, sup)], jbuf)
      pltpu.sync_copy(v_hbm.at[pl.ds(cb, sup)], vbuf)
      descs = [
          pltpu.async_copy(g_hbm.at[ibuf.at[pl.ds(qo, qn)]],
                           sbuf.at[pl.ds(qo, qn)], sem)
          for qo, qn in qs
      ]
      for dsc in descs:
        dsc.wait()
      descs = [
          pltpu.async_copy(h_hbm.at[jbuf.at[pl.ds(qo, qn)]],
                           sbuf.at[pl.ds(qo, qn)], sem, add=True)
          for qo, qn in qs
      ]
      for dsc in descs:
        dsc.wait()

      def nb(b, _):
        sl = pl.ds(b * L, L)
        dg = plsc.load_gather(deg, [ibuf[sl]])
        nbuf[pl.ds(sc * sup + b * L, L)] = jnp.abs(vbuf[sl]) / dg
        return 0

      lax.fori_loop(0, sup // L, nb, 0)
      pltpu.sync_copy(sbuf, s_hbm.at[pl.ds(cb, sup)])
    pltpu.sync_copy(nbuf, n_hbm.at[pl.ds(base, ew)])

  return pl.kernel(
      body,
      out_type=[
          jax.ShapeDtypeStruct((ew * NW, h1), jnp.float32),
          jax.ShapeDtypeStruct((ew * NW,), jnp.float32),
      ],
      mesh=_sc_mesh(),
      compiler_params=pltpu.CompilerParams(
          needs_layout_passes=False, use_tc_tiling_on_sc=False),
      scratch_types=[
          pltpu.VMEM((sup,), jnp.int32),
          pltpu.VMEM((sup,), jnp.int32),
          pltpu.VMEM((sup,), jnp.float32),
          pltpu.VMEM((ew,), jnp.float32),
          pltpu.VMEM((npad,), jnp.float32),
          pltpu.VMEM((npad,), jnp.float32),
          pltpu.VMEM((sup, h1), jnp.float32),
          pltpu.SemaphoreType.DMA,
      ],
  )(i32e, j32e, val, g, h, deg_part)


def _tc_project(f, W1, b1):
  n, d = f.shape
  h1 = W1.shape[1]
  W1cat = jnp.concatenate([W1[:d], W1[d:]], axis=1)
  b1h = (jnp.concatenate([b1, b1]) * 0.5).reshape(1, 2 * h1)

  def body(f_ref, w_ref, b_ref, g_ref, h_ref):
    gh = jnp.dot(f_ref[...], w_ref[...], preferred_element_type=jnp.float32)
    gh = gh + b_ref[...]
    g_ref[...] = gh[:, :h1]
    h_ref[...] = gh[:, h1:]

  return pl.pallas_call(
      body,
      out_shape=[
          jax.ShapeDtypeStruct((n, h1), jnp.float32),
          jax.ShapeDtypeStruct((n, h1), jnp.float32),
      ],
  )(f, W1cat, b1h)


def _tc_mlp(s, norm, W2, b2, Wc):
  ecnt, h1 = s.shape
  h2 = W2.shape[0]
  be = 6400
  grid = ecnt // be
  r8 = be // 8
  rq = be // 128
  rqp = (rq + 7) // 8 * 8
  s2d = s.reshape(ecnt // 8, 128)
  n2d = norm.reshape(grid, rq, 128)
  n2d = jnp.pad(n2d, ((0, 0), (0, rqp - rq), (0, 0))).reshape(grid * rqp, 128)

  eye8 = jnp.eye(8, dtype=jnp.float32)
  w2big = jnp.kron(eye8, W2[:h1, :])
  bsel = jnp.tile(jnp.kron(eye8, W2[h1:, :].reshape(1, h2)), (16, 1))
  wcd = jnp.tile(jnp.kron(eye8, Wc.reshape(h2, 1)), (1, 16))
  b2big = jnp.tile(b2.reshape(1, h2), (1, 8))
  amat = jnp.kron(jnp.eye(rq, rqp, dtype=jnp.float32),
                  jnp.ones((16, 1), jnp.float32))
  cmat = jnp.kron(jnp.eye(rqp, rq, dtype=jnp.float32),
                  jnp.ones((1, 16), jnp.float32))
  mask = jnp.tile(jnp.kron(jnp.eye(16, dtype=jnp.float32),
                           jnp.ones((1, 8), jnp.float32)), (rq, 1))

  def body(s_ref, n_ref, w2big_ref, bsel_ref, wcd_ref, b2big_ref, amat_ref,
           cmat_ref, mask_ref, e_ref):
    sv = s_ref[...]
    x1 = jnp.where(sv >= 0, sv, 0.1 * sv)
    nt = jnp.dot(amat_ref[...], n_ref[...],
                 preferred_element_type=jnp.float32) * mask_ref[...]
    pre = (jnp.dot(x1, w2big_ref[...], preferred_element_type=jnp.float32)
           + jnp.dot(nt, bsel_ref[...], preferred_element_type=jnp.float32)
           + b2big_ref[...])
    x2 = jnp.where(pre >= 0, pre, 0.1 * pre)
    t2 = jnp.dot(x2, wcd_ref[...],
                 preferred_element_type=jnp.float32) * mask_ref[...]
    e_ref[...] = jnp.dot(cmat_ref[...], t2, preferred_element_type=jnp.float32)

  e2d = pl.pallas_call(
      body,
      grid=(grid,),
      in_specs=[
          pl.BlockSpec((r8, 128), lambda t: (t, 0)),
          pl.BlockSpec((rqp, 128), lambda t: (t, 0)),
          pl.BlockSpec((128, 8 * h2), lambda t: (0, 0)),
          pl.BlockSpec((128, 8 * h2), lambda t: (0, 0)),
          pl.BlockSpec((8 * h2, 128), lambda t: (0, 0)),
          pl.BlockSpec((1, 8 * h2), lambda t: (0, 0)),
          pl.BlockSpec((r8, rqp), lambda t: (0, 0)),
          pl.BlockSpec((rqp, r8), lambda t: (0, 0)),
          pl.BlockSpec((r8, 128), lambda t: (0, 0)),
      ],
      out_specs=pl.BlockSpec((rqp, 128), lambda t: (t, 0)),
      out_shape=jax.ShapeDtypeStruct((grid * rqp, 128), jnp.float32),
  )(s2d, n2d, w2big, bsel, wcd, b2big, amat, cmat, mask)
  return e2d.reshape(grid, rqp, 128)[:, :rq, :].reshape(ecnt)


def kernel(Jt_ind, Jt_val, f, N, W1, b1, W2, b2, Wc, bc):
  del N, bc
  ecnt = Jt_ind.shape[1]
  n = f.shape[0]
  npad = ((n + 255) // 256) * 256
  ew = ecnt // NW
  i32e = Jt_ind[0].astype(jnp.int32)
  j32e = Jt_ind[1].astype(jnp.int32)
  val = Jt_val.astype(jnp.float32)

  deg_part = _sc_segsum_abs(i32e, val, npad, ew)
  g, h = _tc_project(f, W1, b1)
  s, norm = _sc_gather(i32e, j32e, val, g, h, deg_part, npad, ew)
  e = _tc_mlp(s, norm, W2, b2, Wc)
  mx_part = _sc_segmax(i32e, e, npad, ew)
  p, den_part = _sc_exp_segsum(i32e, e, mx_part, npad, ew)
  a = _sc_final(i32e, p, den_part, npad, ew)
  return a


# speedup vs baseline: 40.9718x; 1.1352x over previous
import jax
import jax.numpy as jnp
from jax import lax
from jax.experimental import pallas as pl
from jax.experimental.pallas import tpu as pltpu
from jax.experimental.pallas import tpu_sc as plsc

NC = 2
NS = 16
L = 16
NW = NC * NS

_NEG_INF = float("-inf")


def _take(x, idx):
  return jnp.take_along_axis(x, idx, axis=0, mode="promise_in_bounds")


def _seg_scatter(ref, k, v, op, ident):
  ks, vs = plsc.sort_key_val(k, v)
  lane = lax.iota(jnp.int32, L)
  for s in (1, 2, 4, 8):
    idx = jnp.maximum(lane - s, 0)
    same = jnp.logical_and(_take(ks, idx) == ks, lane >= s)
    vs = op(vs, jnp.where(same, _take(vs, idx), ident))
  nxt = _take(ks, jnp.minimum(lane + 1, L - 1))
  last = jnp.logical_or(nxt != ks, lane == L - 1)
  cur = plsc.load_gather(ref, [ks], mask=last)
  plsc.store_scatter(ref, [ks], op(cur, vs), mask=last)


def _fill(ref, n, value):
  splat = jnp.full((L,), value, jnp.float32)

  def body(u, _):
    ref[pl.ds(u * L, L)] = splat
    return 0

  lax.fori_loop(0, n // L, body, 0)


def _ewise(dst, src, n, op):
  def body(u, _):
    sl = pl.ds(u * L, L)
    dst[sl] = op(dst[sl], src[sl])
    return 0

  lax.fori_loop(0, n // L, body, 0)


def _combine_and_emit(loc, shared, acc, tmp, part_hbm, cid, sid, npad, op):
  s_sz = npad // NS
  pltpu.sync_copy(loc, shared.at[sid])
  plsc.subcore_barrier()
  base = sid * s_sz
  pltpu.sync_copy(shared.at[0, pl.ds(base, s_sz)], acc)
  for t in range(1, NS):
    pltpu.sync_copy(shared.at[t, pl.ds(base, s_sz)], tmp.at[pl.ds(0, s_sz)])
    _ewise(acc, tmp, s_sz, op)
  pltpu.sync_copy(acc, part_hbm.at[pl.ds(cid * npad + base, s_sz)])


def _merge_parts(part_hbm, dst, tmp, npad, op):
  pltpu.sync_copy(part_hbm.at[pl.ds(0, npad)], dst)
  pltpu.sync_copy(part_hbm.at[pl.ds(npad, npad)], tmp)
  _ewise(dst, tmp, npad, op)


def _wid(cid, sid):
  return sid * NC + cid


def _sc_mesh():
  return plsc.VectorSubcoreMesh(core_axis_name="c", subcore_axis_name="s")


def _sc_segsum_abs(i32e, val, npad, ew):
  def body(i_hbm, v_hbm, part_hbm, ibuf, vbuf, loc, acc, tmp, shared):
    cid = lax.axis_index("c")
    sid = lax.axis_index("s")
    base = _wid(cid, sid) * ew
    pltpu.sync_copy(i_hbm.at[pl.ds(base, ew)], ibuf)
    pltpu.sync_copy(v_hbm.at[pl.ds(base, ew)], vbuf)
    _fill(loc, npad, 0.0)

    def eb(b, _):
      sl = pl.ds(b * L, L)
      plsc.addupdate_scatter(loc, [ibuf[sl]], jnp.abs(vbuf[sl]))
      return 0

    lax.fori_loop(0, ew // L, eb, 0)
    _combine_and_emit(loc, shared, acc, tmp, part_hbm, cid, sid, npad, jnp.add)

  return pl.kernel(
      body,
      out_type=jax.ShapeDtypeStruct((NC * npad,), jnp.float32),
      mesh=_sc_mesh(),
      compiler_params=pltpu.CompilerParams(needs_layout_passes=False),
      scratch_types=[
          pltpu.VMEM((ew,), jnp.int32),
          pltpu.VMEM((ew,), jnp.float32),
          pltpu.VMEM((npad,), jnp.float32),
          pltpu.VMEM((npad // NS,), jnp.float32),
          pltpu.VMEM((npad,), jnp.float32),
          pltpu.VMEM_SHARED((NS, npad), jnp.float32),
      ],
  )(i32e, val)


def _sc_segmax(i32e, e, npad, ew):
  def body(i_hbm, e_hbm, part_hbm, ibuf, ebuf, loc, acc, tmp, shared):
    cid = lax.axis_index("c")
    sid = lax.axis_index("s")
    base = _wid(cid, sid) * ew
    pltpu.sync_copy(i_hbm.at[pl.ds(base, ew)], ibuf)
    pltpu.sync_copy(e_hbm.at[pl.ds(base, ew)], ebuf)
    _fill(loc, npad, _NEG_INF)

    def eb(b, _):
      sl = pl.ds(b * L, L)
      _seg_scatter(loc, ibuf[sl], ebuf[sl], jnp.maximum, _NEG_INF)
      return 0

    lax.fori_loop(0, ew // L, eb, 0)
    _combine_and_emit(loc, shared, acc, tmp, part_hbm, cid, sid, npad,
                      jnp.maximum)

  return pl.kernel(
      body,
      out_type=jax.ShapeDtypeStruct((NC * npad,), jnp.float32),
      mesh=_sc_mesh(),
      compiler_params=pltpu.CompilerParams(needs_layout_passes=False),
      scratch_types=[
          pltpu.VMEM((ew,), jnp.int32),
          pltpu.VMEM((ew,), jnp.float32),
          pltpu.VMEM((npad,), jnp.float32),
          pltpu.VMEM((npad // NS,), jnp.float32),
          pltpu.VMEM((npad,), jnp.float32),
          pltpu.VMEM_SHARED((NS, npad), jnp.float32),
      ],
  )(i32e, e)


def _sc_exp_segsum(i32e, e, mx_part, npad, ew):
  def body(i_hbm, e_hbm, mx_hbm, p_hbm, part_hbm, ibuf, ebuf, m, loc, acc,
           tmp, shared):
    cid = lax.axis_index("c")
    sid = lax.axis_index("s")
    base = _wid(cid, sid) * ew
    _merge_parts(mx_hbm, m, tmp, npad, jnp.maximum)
    pltpu.sync_copy(i_hbm.at[pl.ds(base, ew)], ibuf)
    pltpu.sync_copy(e_hbm.at[pl.ds(base, ew)], ebuf)
    _fill(loc, npad, 0.0)

    def eb(b, _):
      sl = pl.ds(b * L, L)
      k = ibuf[sl]
      mg = plsc.load_gather(m, [k])
      p = jnp.exp(ebuf[sl] - mg)
      ebuf[sl] = p
      plsc.addupdate_scatter(loc, [k], p)
      return 0

    lax.fori_loop(0, ew // L, eb, 0)
    pltpu.sync_copy(ebuf, p_hbm.at[pl.ds(base, ew)])
    _combine_and_emit(loc, shared, acc, tmp, part_hbm, cid, sid, npad, jnp.add)

  return pl.kernel(
      body,
      out_type=[
          jax.ShapeDtypeStruct((ew * NW,), jnp.float32),
          jax.ShapeDtypeStruct((NC * npad,), jnp.float32),
      ],
      mesh=_sc_mesh(),
      compiler_params=pltpu.CompilerParams(needs_layout_passes=False),
      scratch_types=[
          pltpu.VMEM((ew,), jnp.int32),
          pltpu.VMEM((ew,), jnp.float32),
          pltpu.VMEM((npad,), jnp.float32),
          pltpu.VMEM((npad,), jnp.float32),
          pltpu.VMEM((npad // NS,), jnp.float32),
          pltpu.VMEM((npad,), jnp.float32),
          pltpu.VMEM_SHARED((NS, npad), jnp.float32),
      ],
  )(i32e, e, mx_part)


def _sc_final(i32e, p, den_part, npad, ew):
  def body(i_hbm, p_hbm, den_hbm, a_hbm, ibuf, pbuf, d, tmp):
    cid = lax.axis_index("c")
    sid = lax.axis_index("s")
    base = _wid(cid, sid) * ew
    _merge_parts(den_hbm, d, tmp, npad, jnp.add)
    pltpu.sync_copy(i_hbm.at[pl.ds(base, ew)], ibuf)
    pltpu.sync_copy(p_hbm.at[pl.ds(base, ew)], pbuf)

    def eb(b, _):
      sl = pl.ds(b * L, L)
      dg = plsc.load_gather(d, [ibuf[sl]])
      pbuf[sl] = pbuf[sl] / dg
      return 0

    lax.fori_loop(0, ew // L, eb, 0)
    pltpu.sync_copy(pbuf, a_hbm.at[pl.ds(base, ew)])

  return pl.kernel(
      body,
      out_type=jax.ShapeDtypeStruct((ew * NW,), jnp.float32),
      mesh=_sc_mesh(),
      compiler_params=pltpu.CompilerParams(needs_layout_passes=False),
      scratch_types=[
          pltpu.VMEM((ew,), jnp.int32),
          pltpu.VMEM((ew,), jnp.float32),
          pltpu.VMEM((npad,), jnp.float32),
          pltpu.VMEM((npad,), jnp.float32),
      ],
  )(i32e, p, den_part)


def _sc_gather(i32e, j32e, val, g, h, deg_part, npad, ew):
  h1 = g.shape[1]
  sup = 2000
  nsup = ew // sup
  qs = []
  off = 0
  while off < sup:
    qn = min(128, sup - off)
    qs.append((off, qn))
    off += qn

  def body(i_hbm, j_hbm, v_hbm, g_hbm, h_hbm, dp_hbm, s_hbm, n_hbm,
           ibuf0, jbuf0, vbuf0, ibuf1, jbuf1, vbuf1, nbuf, deg, tmp,
           sbuf0, sbuf1, gsem, hsem, wsem):
    cid = lax.axis_index("c")
    sid = lax.axis_index("s")
    base = _wid(cid, sid) * ew
    _merge_parts(dp_hbm, deg, tmp, npad, jnp.add)
    ib = (ibuf0, ibuf1)
    jb = (jbuf0, jbuf1)
    vb = (vbuf0, vbuf1)
    sb = (sbuf0, sbuf1)

    def stage(t):
      cb = base + t * sup
      pltpu.sync_copy(i_hbm.at[pl.ds(cb, sup)], ib[t % 2])
      pltpu.sync_copy(j_hbm.at[pl.ds(cb, sup)], jb[t % 2])
      pltpu.sync_copy(v_hbm.at[pl.ds(cb, sup)], vb[t % 2])

    def issue_g(t):
      return [pltpu.async_copy(g_hbm.at[ib[t % 2].at[pl.ds(qo, qn)]],
                               sb[t % 2].at[pl.ds(qo, qn)], gsem)
              for qo, qn in qs]

    def issue_h(t):
      return [pltpu.async_copy(h_hbm.at[jb[t % 2].at[pl.ds(qo, qn)]],
                               sb[t % 2].at[pl.ds(qo, qn)], hsem, add=True)
              for qo, qn in qs]

    def norm_compute(t):
      def nb(b, _):
        sl = pl.ds(b * L, L)
        dg = plsc.load_gather(deg, [ib[t % 2][sl]])
        nbuf[pl.ds(t * sup + b * L, L)] = jnp.abs(vb[t % 2][sl]) / dg
        return 0
      lax.fori_loop(0, sup // L, nb, 0)

    stage(0)
    gd = issue_g(0)
    wd = [None, None]
    for t in range(nsup):
      for d in gd:
        d.wait()
      hd = issue_h(t)
      if t + 1 < nsup:
        stage(t + 1)
        if wd[(t + 1) % 2] is not None:
          wd[(t + 1) % 2].wait()
          wd[(t + 1) % 2] = None
        gd = issue_g(t + 1)
      for d in hd:
        d.wait()
      norm_compute(t)
      wd[t % 2] = pltpu.async_copy(
          sb[t % 2], s_hbm.at[pl.ds(base + t * sup, sup)], wsem)
    for d in wd:
      if d is not None:
        d.wait()
    pltpu.sync_copy(nbuf, n_hbm.at[pl.ds(base, ew)])

  return pl.kernel(
      body,
      out_type=[
          jax.ShapeDtypeStruct((ew * NW, h1), jnp.float32),
          jax.ShapeDtypeStruct((ew * NW,), jnp.float32),
      ],
      mesh=_sc_mesh(),
      compiler_params=pltpu.CompilerParams(
          needs_layout_passes=False, use_tc_tiling_on_sc=False),
      scratch_types=[
          pltpu.VMEM((sup,), jnp.int32),
          pltpu.VMEM((sup,), jnp.int32),
          pltpu.VMEM((sup,), jnp.float32),
          pltpu.VMEM((sup,), jnp.int32),
          pltpu.VMEM((sup,), jnp.int32),
          pltpu.VMEM((sup,), jnp.float32),
          pltpu.VMEM((ew,), jnp.float32),
          pltpu.VMEM((npad,), jnp.float32),
          pltpu.VMEM((npad,), jnp.float32),
          pltpu.VMEM((sup, h1), jnp.float32),
          pltpu.VMEM((sup, h1), jnp.float32),
          pltpu.SemaphoreType.DMA,
          pltpu.SemaphoreType.DMA,
          pltpu.SemaphoreType.DMA,
      ],
  )(i32e, j32e, val, g, h, deg_part)


def _tc_project(f, W1, b1):
  n, d = f.shape
  h1 = W1.shape[1]
  W1cat = jnp.concatenate([W1[:d], W1[d:]], axis=1)
  b1h = (jnp.concatenate([b1, b1]) * 0.5).reshape(1, 2 * h1)

  def body(f_ref, w_ref, b_ref, g_ref, h_ref):
    gh = jnp.dot(f_ref[...], w_ref[...], preferred_element_type=jnp.float32)
    gh = gh + b_ref[...]
    g_ref[...] = gh[:, :h1]
    h_ref[...] = gh[:, h1:]

  return pl.pallas_call(
      body,
      out_shape=[
          jax.ShapeDtypeStruct((n, h1), jnp.float32),
          jax.ShapeDtypeStruct((n, h1), jnp.float32),
      ],
  )(f, W1cat, b1h)


def _tc_mlp(s, norm, W2, b2, Wc):
  ecnt, h1 = s.shape
  h2 = W2.shape[0]
  be = 12800
  grid = ecnt // be
  r8 = be // 8
  rq = be // 128
  rqp = (rq + 7) // 8 * 8
  s2d = s.reshape(ecnt // 8, 128)
  n2d = norm.reshape(grid, rq, 128)
  n2d = jnp.pad(n2d, ((0, 0), (0, rqp - rq), (0, 0))).reshape(grid * rqp, 128)

  eye8 = jnp.eye(8, dtype=jnp.float32)
  w2big = jnp.kron(eye8, W2[:h1, :])
  bsel = jnp.tile(jnp.kron(eye8, W2[h1:, :].reshape(1, h2)), (16, 1))
  wcd = jnp.tile(jnp.kron(eye8, Wc.reshape(h2, 1)), (1, 16))
  b2big = jnp.tile(b2.reshape(1, h2), (1, 8))
  amat = jnp.kron(jnp.eye(rq, rqp, dtype=jnp.float32),
                  jnp.ones((16, 1), jnp.float32))
  cmat = jnp.kron(jnp.eye(rqp, rq, dtype=jnp.float32),
                  jnp.ones((1, 16), jnp.float32))
  mask = jnp.tile(jnp.kron(jnp.eye(16, dtype=jnp.float32),
                           jnp.ones((1, 8), jnp.float32)), (rq, 1))

  def body(s_ref, n_ref, w2big_ref, bsel_ref, wcd_ref, b2big_ref, amat_ref,
           cmat_ref, mask_ref, e_ref):
    sv = s_ref[...]
    x1 = jnp.where(sv >= 0, sv, 0.1 * sv)
    nt = jnp.dot(amat_ref[...], n_ref[...],
                 preferred_element_type=jnp.float32) * mask_ref[...]
    pre = (jnp.dot(x1, w2big_ref[...], preferred_element_type=jnp.float32)
           + jnp.dot(nt, bsel_ref[...], preferred_element_type=jnp.float32)
           + b2big_ref[...])
    x2 = jnp.where(pre >= 0, pre, 0.1 * pre)
    t2 = jnp.dot(x2, wcd_ref[...],
                 preferred_element_type=jnp.float32) * mask_ref[...]
    e_ref[...] = jnp.dot(cmat_ref[...], t2, preferred_element_type=jnp.float32)

  e2d = pl.pallas_call(
      body,
      grid=(grid,),
      in_specs=[
          pl.BlockSpec((r8, 128), lambda t: (t, 0)),
          pl.BlockSpec((rqp, 128), lambda t: (t, 0)),
          pl.BlockSpec((128, 8 * h2), lambda t: (0, 0)),
          pl.BlockSpec((128, 8 * h2), lambda t: (0, 0)),
          pl.BlockSpec((8 * h2, 128), lambda t: (0, 0)),
          pl.BlockSpec((1, 8 * h2), lambda t: (0, 0)),
          pl.BlockSpec((r8, rqp), lambda t: (0, 0)),
          pl.BlockSpec((rqp, r8), lambda t: (0, 0)),
          pl.BlockSpec((r8, 128), lambda t: (0, 0)),
      ],
      out_specs=pl.BlockSpec((rqp, 128), lambda t: (t, 0)),
      out_shape=jax.ShapeDtypeStruct((grid * rqp, 128), jnp.float32),
  )(s2d, n2d, w2big, bsel, wcd, b2big, amat, cmat, mask)
  return e2d.reshape(grid, rqp, 128)[:, :rq, :].reshape(ecnt)


def kernel(Jt_ind, Jt_val, f, N, W1, b1, W2, b2, Wc, bc):
  del N, bc
  ecnt = Jt_ind.shape[1]
  n = f.shape[0]
  npad = ((n + 255) // 256) * 256
  ew = ecnt // NW
  i32e = Jt_ind[0].astype(jnp.int32)
  j32e = Jt_ind[1].astype(jnp.int32)
  val = Jt_val.astype(jnp.float32)

  deg_part = _sc_segsum_abs(i32e, val, npad, ew)
  g, h = _tc_project(f, W1, b1)
  s, norm = _sc_gather(i32e, j32e, val, g, h, deg_part, npad, ew)
  e = _tc_mlp(s, norm, W2, b2, Wc)
  mx_part = _sc_segmax(i32e, e, npad, ew)
  p, den_part = _sc_exp_segsum(i32e, e, mx_part, npad, ew)
  a = _sc_final(i32e, p, den_part, npad, ew)
  return a
